# baseline (speedup 1.0000x reference)
import functools

import jax
import jax.numpy as jnp
from jax import lax
from jax.experimental import pallas as pl
from jax.experimental.pallas import tpu as pltpu
from jax.experimental.pallas import tpu_sc as plsc

NC, NS = 2, 16
NW = NC * NS
B, L, D = 4096, 200, 128
N = B * L
N_SC = 573440
N_TC = N - N_SC
ROWS_PER_W = N_SC // NW
NDAY, NWEEK, NMONTH = 32, 7, 13
DSTRIDE = 96
COMB = NDAY * DSTRIDE
DPT = NDAY // NS
SLAB = DPT * DSTRIDE

IDXBLK = 2560
NBLK = ROWS_PER_W // IDXBLK
GCH = 128
CPB = IDXBLK // GCH
NCH = ROWS_PER_W // GCH

BT = 512
NBT = N_TC // BT

_mesh = plsc.VectorSubcoreMesh(
    core_axis_name="c", subcore_axis_name="s", num_cores=NC, num_subcores=NS)


@functools.partial(
    pl.kernel,
    out_type=jax.ShapeDtypeStruct((N_SC, D), jnp.float32),
    mesh=_mesh,
    scratch_types=[
        pltpu.VMEM((DPT * D,), jnp.float32),
        pltpu.VMEM((NWEEK * D,), jnp.float32),
        pltpu.VMEM((NMONTH * D,), jnp.float32),
        pltpu.VMEM((DSTRIDE, D), jnp.float32),
        pltpu.VMEM_SHARED((COMB, D), jnp.float32),
        pltpu.VMEM((2, IDXBLK), jnp.int32),
        pltpu.VMEM((2, IDXBLK), jnp.int32),
        pltpu.VMEM((2, IDXBLK), jnp.int32),
        pltpu.VMEM((2, CPB, 128), jnp.int32),
        pltpu.VMEM((GCH, D), jnp.float32),
        pltpu.VMEM((GCH, D), jnp.float32),
        pltpu.VMEM((GCH, D), jnp.float32),
        pltpu.VMEM((GCH, D), jnp.float32),
        pltpu.SemaphoreType.DMA,
        pltpu.SemaphoreType.DMA,
        pltpu.SemaphoreType.DMA,
        pltpu.SemaphoreType.DMA,
        pltpu.SemaphoreType.DMA,
        pltpu.SemaphoreType.DMA,
        pltpu.SemaphoreType.DMA,
        pltpu.SemaphoreType.DMA,
        pltpu.SemaphoreType.DMA,
        pltpu.SemaphoreType.DMA,
    ],
)
def _temporal_embed(day_h, week_h, month_h, dt_h, wt_h, mt_h, out_h,
                    dtv, wtv, mtv, slab, comb_sh, di, wi, mi, cbuf,
                    rows0, rows1, rows2, rows3, is0, is1,
                    gs0, gs1, gs2, gs3, ws0, ws1, ws2, ws3):
    sid = lax.axis_index("s")
    wid = sid * NC + lax.axis_index("c")
    base0 = wid * ROWS_PER_W

    pltpu.sync_copy(dt_h.at[pl.ds(sid * (DPT * D), DPT * D)], dtv)
    pltpu.sync_copy(wt_h, wtv)
    pltpu.sync_copy(mt_h, mtv)
    for dd in range(DPT):
        for w in range(NWEEK):
            dw = [dtv[pl.ds(dd * D + j * 16, 16)] + wtv[pl.ds(w * D + j * 16, 16)]
                  for j in range(D // 16)]
            for m in range(NMONTH):
                r = w * NMONTH + m
                for j in range(D // 16):
                    slab[r, pl.ds(j * 16, 16)] = dw[j] + mtv[pl.ds(m * D + j * 16, 16)]
        pltpu.sync_copy(
            slab, comb_sh.at[pl.ds(sid * SLAB + dd * DSTRIDE, DSTRIDE)])
    plsc.subcore_barrier()

    bufs = (rows0, rows1, rows2, rows3)
    gsems = (gs0, gs1, gs2, gs3)
    wsems = (ws0, ws1, ws2, ws3)
    isems = (is0, is1)

    def idx_copies(t, sl):
        boff = base0 + t * IDXBLK
        sem = isems[sl]
        return [
            pltpu.make_async_copy(day_h.at[pl.ds(boff, IDXBLK)], di.at[sl], sem),
            pltpu.make_async_copy(week_h.at[pl.ds(boff, IDXBLK)], wi.at[sl], sem),
            pltpu.make_async_copy(month_h.at[pl.ds(boff, IDXBLK)], mi.at[sl], sem),
        ]

    def compute_cidx(sl):
        def grp(g, c2):
            s = pl.ds(g * 16, 16)
            d = jnp.clip(di[sl, s], 0, NDAY - 1)
            w = jnp.clip(wi[sl, s], 0, NWEEK - 1)
            m = jnp.clip(mi[sl, s], 0, NMONTH - 1)
            cbuf[sl, g // 8, pl.ds((g % 8) * 16, 16)] = (
                d * DSTRIDE + w * NMONTH + m)
            return c2

        lax.fori_loop(0, IDXBLK // 16, grp, 0)

    def gather(t, cl, rbuf, sem):
        return pltpu.make_async_copy(
            comb_sh.at[cbuf.at[t % 2, cl]], rbuf, sem)

    def write(c, rbuf, sem):
        return pltpu.make_async_copy(
            rbuf, out_h.at[pl.ds(base0 + c * GCH, GCH)], sem)

    for cp in idx_copies(0, 0):
        cp.start()
    for cp in idx_copies(0, 0):
        cp.wait()
    compute_cidx(0)
    for cp in idx_copies(1, 1):
        cp.start()
    gather(0, 0, bufs[0], gsems[0]).start()
    gather(0, 1, bufs[1], gsems[1]).start()

    for t in range(NBLK):
        if t + 1 < NBLK:
            for cp in idx_copies(t + 1, (t + 1) % 2):
                cp.wait()
            compute_cidx((t + 1) % 2)
        if t + 2 < NBLK:
            for cp in idx_copies(t + 2, t % 2):
                cp.start()

        def step(q, carry):
            for j in range(4):
                cl = 4 * q + j
                c = t * CPB + cl
                jn = (j + 2) % 4
                gather(t, cl, bufs[j], gsems[j]).wait()
                write(c, bufs[j], wsems[j]).start()

                @pl.when(c + 2 >= 4)
                def _():
                    write(c - 2, bufs[jn], wsems[jn]).wait()
                gather(t, cl + 2, bufs[jn], gsems[jn]).start()
            return carry

        lax.fori_loop(0, 4, step, 0)

        for cl in (16, 17, 18, 19):
            jb = cl % 4
            c = t * CPB + cl
            gather(t, cl, bufs[jb], gsems[jb]).wait()
            write(c, bufs[jb], wsems[jb]).start()
            if cl < 18:
                jn = (jb + 2) % 4
                write(c - 2, bufs[jn], wsems[jn]).wait()
                gather(t, cl + 2, bufs[jn], gsems[jn]).start()
            elif t + 1 < NBLK:
                jn = (jb + 2) % 4
                write(c - 2, bufs[jn], wsems[jn]).wait()
                gather(t + 1, cl - 18, bufs[jn], gsems[jn]).start()

    for j in range(4):
        write(NCH - 4 + j, bufs[j], wsems[j]).wait()


def _onehot_body(day_ref, week_ref, month_ref, tab_ref, out_ref):
    lanes = lax.broadcasted_iota(jnp.int32, (BT, 64), 1)
    d = jnp.clip(day_ref[...], 0, NDAY - 1)
    w = jnp.clip(week_ref[...], 0, NWEEK - 1) + NDAY
    m = jnp.clip(month_ref[...], 0, NMONTH - 1) + (NDAY + NWEEK)
    oh = (jnp.where(d == lanes, 1.0, 0.0)
          + jnp.where(w == lanes, 1.0, 0.0)
          + jnp.where(m == lanes, 1.0, 0.0))
    out_ref[...] = jnp.dot(oh, tab_ref[...],
                           preferred_element_type=jnp.float32)


_onehot_tc = pl.pallas_call(
    _onehot_body,
    out_shape=jax.ShapeDtypeStruct((N_TC, D), jnp.float32),
    grid=(NBT,),
    in_specs=[
        pl.BlockSpec((BT, 1), lambda i: (i, 0)),
        pl.BlockSpec((BT, 1), lambda i: (i, 0)),
        pl.BlockSpec((BT, 1), lambda i: (i, 0)),
        pl.BlockSpec((64, D), lambda i: (0, 0)),
    ],
    out_specs=pl.BlockSpec((BT, D), lambda i: (i, 0)),
)


def kernel(day, week, month, day_table, week_table, month_table):
    day = day.reshape(N).astype(jnp.int32)
    week = week.reshape(N).astype(jnp.int32)
    month = month.reshape(N).astype(jnp.int32)
    out_sc = _temporal_embed(
        day[:N_SC], week[:N_SC], month[:N_SC], day_table.reshape(-1),
        week_table.reshape(-1), month_table.reshape(-1))
    stacked = jnp.zeros((64, D), jnp.float32)
    stacked = stacked.at[:NDAY].set(day_table)
    stacked = stacked.at[NDAY:NDAY + NWEEK].set(week_table)
    stacked = stacked.at[NDAY + NWEEK:NDAY + NWEEK + NMONTH].set(month_table)
    out_tc = _onehot_tc(day[N_SC:, None], week[N_SC:, None],
                        month[N_SC:, None], stacked)
    return jnp.concatenate([out_sc, out_tc], axis=0).reshape(B, L, D)


# speedup vs baseline: 4.2639x; 4.2639x over previous
import functools

import jax
import jax.numpy as jnp
from jax import lax
from jax.experimental import pallas as pl
from jax.experimental.pallas import tpu as pltpu
from jax.experimental.pallas import tpu_sc as plsc

NC, NS = 2, 16
NW = NC * NS
B, L, D = 4096, 200, 128
N = B * L
ROWS_PER_W = N // NW
NDAY, NWEEK, NMONTH = 32, 7, 13
DSTRIDE = 96
COMB = NDAY * DSTRIDE
DPT = NDAY // NS
SLAB = DPT * DSTRIDE

IDXBLK = 2560
NBLK = ROWS_PER_W // IDXBLK
GCH = 128
CPB = IDXBLK // GCH
NCH = ROWS_PER_W // GCH

_mesh = plsc.VectorSubcoreMesh(
    core_axis_name="c", subcore_axis_name="s", num_cores=NC, num_subcores=NS)


@functools.partial(
    pl.kernel,
    out_type=jax.ShapeDtypeStruct((N, D), jnp.float32),
    mesh=_mesh,
    scratch_types=[
        pltpu.VMEM((DPT * D,), jnp.float32),
        pltpu.VMEM((NWEEK * D,), jnp.float32),
        pltpu.VMEM((NMONTH * D,), jnp.float32),
        pltpu.VMEM((DSTRIDE, D), jnp.float32),
        pltpu.VMEM_SHARED((COMB, D), jnp.float32),
        pltpu.VMEM((2, IDXBLK), jnp.int32),
        pltpu.VMEM((2, IDXBLK), jnp.int32),
        pltpu.VMEM((2, IDXBLK), jnp.int32),
        pltpu.VMEM((2, CPB, 128), jnp.int32),
        pltpu.VMEM((GCH, D), jnp.float32),
        pltpu.VMEM((GCH, D), jnp.float32),
        pltpu.VMEM((GCH, D), jnp.float32),
        pltpu.VMEM((GCH, D), jnp.float32),
        pltpu.SemaphoreType.DMA,
        pltpu.SemaphoreType.DMA,
        pltpu.SemaphoreType.DMA,
        pltpu.SemaphoreType.DMA,
        pltpu.SemaphoreType.DMA,
        pltpu.SemaphoreType.DMA,
        pltpu.SemaphoreType.DMA,
        pltpu.SemaphoreType.DMA,
        pltpu.SemaphoreType.DMA,
        pltpu.SemaphoreType.DMA,
    ],
)
def _temporal_embed(day_h, week_h, month_h, dt_h, wt_h, mt_h, out_h,
                    dtv, wtv, mtv, slab, comb_sh, di, wi, mi, cbuf,
                    rows0, rows1, rows2, rows3, is0, is1,
                    gs0, gs1, gs2, gs3, ws0, ws1, ws2, ws3):
    sid = lax.axis_index("s")
    wid = sid * NC + lax.axis_index("c")
    base0 = wid * ROWS_PER_W

    pltpu.sync_copy(dt_h.at[pl.ds(sid * (DPT * D), DPT * D)], dtv)
    pltpu.sync_copy(wt_h, wtv)
    pltpu.sync_copy(mt_h, mtv)
    for dd in range(DPT):
        for w in range(NWEEK):
            dw = [dtv[pl.ds(dd * D + j * 16, 16)] + wtv[pl.ds(w * D + j * 16, 16)]
                  for j in range(D // 16)]
            for m in range(NMONTH):
                r = w * NMONTH + m
                for j in range(D // 16):
                    slab[r, pl.ds(j * 16, 16)] = dw[j] + mtv[pl.ds(m * D + j * 16, 16)]
        pltpu.sync_copy(
            slab, comb_sh.at[pl.ds(sid * SLAB + dd * DSTRIDE, DSTRIDE)])
    plsc.subcore_barrier()

    bufs = (rows0, rows1, rows2, rows3)
    gsems = (gs0, gs1, gs2, gs3)
    wsems = (ws0, ws1, ws2, ws3)
    isems = (is0, is1)

    def idx_copies(t, sl):
        boff = base0 + t * IDXBLK
        sem = isems[sl]
        return [
            pltpu.make_async_copy(day_h.at[pl.ds(boff, IDXBLK)], di.at[sl], sem),
            pltpu.make_async_copy(week_h.at[pl.ds(boff, IDXBLK)], wi.at[sl], sem),
            pltpu.make_async_copy(month_h.at[pl.ds(boff, IDXBLK)], mi.at[sl], sem),
        ]

    def compute_cidx(sl):
        def grp(g, c2):
            s = pl.ds(g * 16, 16)
            d = jnp.clip(di[sl, s], 0, NDAY - 1)
            w = jnp.clip(wi[sl, s], 0, NWEEK - 1)
            m = jnp.clip(mi[sl, s], 0, NMONTH - 1)
            cbuf[sl, g // 8, pl.ds((g % 8) * 16, 16)] = (
                d * DSTRIDE + w * NMONTH + m)
            return c2

        lax.fori_loop(0, IDXBLK // 16, grp, 0)

    def gather(t, cl, rbuf, sem):
        return pltpu.make_async_copy(
            comb_sh.at[cbuf.at[t % 2, cl]], rbuf, sem)

    def write(c, rbuf, sem):
        return pltpu.make_async_copy(
            rbuf, out_h.at[pl.ds(base0 + c * GCH, GCH)], sem)

    for cp in idx_copies(0, 0):
        cp.start()
    for cp in idx_copies(0, 0):
        cp.wait()
    compute_cidx(0)
    for cp in idx_copies(1, 1):
        cp.start()
    gather(0, 0, bufs[0], gsems[0]).start()
    gather(0, 1, bufs[1], gsems[1]).start()

    for t in range(NBLK):
        if t + 1 < NBLK:
            for cp in idx_copies(t + 1, (t + 1) % 2):
                cp.wait()
            compute_cidx((t + 1) % 2)
        if t + 2 < NBLK:
            for cp in idx_copies(t + 2, t % 2):
                cp.start()

        def step(q, carry):
            for j in range(4):
                cl = 4 * q + j
                c = t * CPB + cl
                jn = (j + 2) % 4
                gather(t, cl, bufs[j], gsems[j]).wait()
                write(c, bufs[j], wsems[j]).start()

                @pl.when(c + 2 >= 4)
                def _():
                    write(c - 2, bufs[jn], wsems[jn]).wait()
                gather(t, cl + 2, bufs[jn], gsems[jn]).start()
            return carry

        lax.fori_loop(0, 4, step, 0)

        for cl in (16, 17, 18, 19):
            jb = cl % 4
            c = t * CPB + cl
            gather(t, cl, bufs[jb], gsems[jb]).wait()
            write(c, bufs[jb], wsems[jb]).start()
            if cl < 18:
                jn = (jb + 2) % 4
                write(c - 2, bufs[jn], wsems[jn]).wait()
                gather(t, cl + 2, bufs[jn], gsems[jn]).start()
            elif t + 1 < NBLK:
                jn = (jb + 2) % 4
                write(c - 2, bufs[jn], wsems[jn]).wait()
                gather(t + 1, cl - 18, bufs[jn], gsems[jn]).start()

    for j in range(4):
        write(NCH - 4 + j, bufs[j], wsems[j]).wait()


def kernel(day, week, month, day_table, week_table, month_table):
    day = day.reshape(N).astype(jnp.int32)
    week = week.reshape(N).astype(jnp.int32)
    month = month.reshape(N).astype(jnp.int32)
    out = _temporal_embed(day, week, month, day_table.reshape(-1),
                          week_table.reshape(-1), month_table.reshape(-1))
    return out.reshape(B, L, D)
